# baseline (speedup 1.0000x reference)
import functools

import jax
import jax.numpy as jnp
from jax import lax
from jax.experimental import pallas as pl
from jax.experimental.pallas import tpu as pltpu
from jax.experimental.pallas import tpu_sc as plsc

N = 10000
D = 128
H = 128
E = 320000

NC = 2
NS = 16
L = 16
NW = NC * NS

CHUNK = 128
CPT = 80
EPT = CPT * CHUNK
E_PAD = NW * EPT
PAD = E_PAD - E

ACC_ROWS = 10240
RPT = ACC_ROWS // NS
DUMMY_SPREAD = ACC_ROWS - N
DEG_W = 16

_mesh = plsc.VectorSubcoreMesh(core_axis_name="c", subcore_axis_name="s")


@functools.partial(
    pl.kernel,
    mesh=_mesh,
    out_type=jax.ShapeDtypeStruct((NC, ACC_ROWS, DEG_W), jnp.float32),
    scratch_types=[
        pltpu.VMEM((CPT, CHUNK), jnp.int32),
        pltpu.VMEM((CHUNK,), jnp.int32),
        pltpu.VMEM((CHUNK, DEG_W), jnp.float32),
        pltpu.VMEM_SHARED((ACC_ROWS, DEG_W), jnp.float32),
    ],
)
def _deg_kernel(pidx_hbm, out_hbm, pidx_v, didx_v, ones_v, acc_sh):
    c = lax.axis_index("c")
    s = lax.axis_index("s")
    w = c * NS + s

    pltpu.sync_copy(pidx_hbm.at[w], pidx_v)

    @pl.loop(0, CHUNK)
    def _(r):
        ones_v[r, pl.ds(0, L)] = jnp.zeros((L,), jnp.float32)

    @pl.loop(0, RPT, step=CHUNK)
    def _(r0):
        pltpu.sync_copy(ones_v, acc_sh.at[pl.ds(s * RPT + r0, CHUNK)])

    @pl.loop(0, CHUNK)
    def _(r):
        ones_v[r, pl.ds(0, L)] = jnp.ones((L,), jnp.float32)

    plsc.subcore_barrier()

    @pl.loop(0, CPT)
    def _(i):
        @pl.loop(0, CHUNK, step=L)
        def _(k):
            packed = pidx_v[i, pl.ds(k, L)]
            didx_v[pl.ds(k, L)] = lax.shift_right_logical(packed, 16)

        pltpu.sync_copy(ones_v, acc_sh.at[didx_v], add=True)

    plsc.subcore_barrier()
    pltpu.sync_copy(
        acc_sh.at[pl.ds(s * RPT, RPT)], out_hbm.at[c, pl.ds(s * RPT, RPT)]
    )


@functools.partial(
    pl.kernel,
    mesh=_mesh,
    out_type=jax.ShapeDtypeStruct((NC, ACC_ROWS, H), jnp.float32),
    scratch_types=[
        pltpu.VMEM((CPT, CHUNK), jnp.int32),
        pltpu.VMEM((CHUNK,), jnp.int32),
        pltpu.VMEM((CHUNK,), jnp.int32),
        pltpu.VMEM((CHUNK,), jnp.int32),
        pltpu.VMEM((CHUNK,), jnp.int32),
        pltpu.VMEM((CHUNK, H), jnp.float32),
        pltpu.VMEM((CHUNK, H), jnp.float32),
        pltpu.VMEM_SHARED((ACC_ROWS, H), jnp.float32),
        pltpu.SemaphoreType.DMA,
        pltpu.SemaphoreType.DMA,
    ],
)
def _segsum_kernel(pidx_hbm, g_hbm, out_hbm, pidx_v, sidx0_v, sidx1_v,
                   didx0_v, didx1_v, rows0_v, rows1_v, acc_sh, sem0, sem1):
    c = lax.axis_index("c")
    s = lax.axis_index("s")
    w = c * NS + s
    sidx = (sidx0_v, sidx1_v)
    didx = (didx0_v, didx1_v)
    rows = (rows0_v, rows1_v)
    sems = (sem0, sem1)

    pltpu.sync_copy(pidx_hbm.at[w], pidx_v)

    def unpack(cch, b):
        @pl.loop(0, CHUNK, step=L)
        def _(k):
            packed = pidx_v[cch, pl.ds(k, L)]
            sidx[b][pl.ds(k, L)] = lax.bitwise_and(packed, 0xFFFF)
            didx[b][pl.ds(k, L)] = lax.shift_right_logical(packed, 16)

    @pl.loop(0, CHUNK)
    def _(r):
        @pl.loop(0, H, step=L)
        def _(c0):
            rows0_v[r, pl.ds(c0, L)] = jnp.zeros((L,), jnp.float32)

    @pl.loop(0, RPT, step=CHUNK)
    def _(r0):
        pltpu.sync_copy(rows0_v, acc_sh.at[pl.ds(s * RPT + r0, CHUNK)])

    plsc.subcore_barrier()

    unpack(0, 0)
    pltpu.async_copy(g_hbm.at[sidx[0]], rows0_v, sem0)

    @pl.loop(0, CPT, step=2)
    def _(i):
        for b in range(2):
            cch = i + b
            pltpu.make_async_copy(
                g_hbm.at[sidx[b]], rows[b], sems[b]).wait()
            @pl.when(cch + 1 < CPT)
            def _():
                unpack(cch + 1, 1 - b)
                pltpu.async_copy(
                    g_hbm.at[sidx[1 - b]], rows[1 - b], sems[1 - b])
            pltpu.sync_copy(rows[b], acc_sh.at[didx[b]], add=True)

    plsc.subcore_barrier()
    pltpu.sync_copy(
        acc_sh.at[pl.ds(s * RPT, RPT)], out_hbm.at[c, pl.ds(s * RPT, RPT)]
    )


def _prologue_body(degp_ref, x_ref, w0_ref, dis_ref, g0_ref):
    degp = degp_ref[...]
    deg = jnp.sum(degp[0, :N, :] + degp[1, :N, :], axis=1) + 1.0
    dis = lax.rsqrt(deg)[:, None]
    h = jnp.dot(x_ref[...], w0_ref[...], preferred_element_type=jnp.float32)
    dis_ref[...] = dis
    g0_ref[...] = h * dis


_prologue = pl.pallas_call(
    _prologue_body,
    out_shape=[
        jax.ShapeDtypeStruct((N, 1), jnp.float32),
        jax.ShapeDtypeStruct((N, H), jnp.float32),
    ],
)


def _mid_body(p_ref, g0_ref, dis_ref, b0_ref, w1_ref, g1_ref):
    S = p_ref[0, :N, :] + p_ref[1, :N, :] + g0_ref[...]
    dis = dis_ref[...]
    h1 = jnp.maximum(S * dis + b0_ref[...], 0.0)
    g1_ref[...] = (
        jnp.dot(h1, w1_ref[...], preferred_element_type=jnp.float32) * dis
    )


_mid = pl.pallas_call(
    _mid_body,
    out_shape=jax.ShapeDtypeStruct((N, H), jnp.float32),
)


def _final_body(p_ref, g1_ref, dis_ref, b1_ref, out_ref):
    S = p_ref[0, :N, :] + p_ref[1, :N, :] + g1_ref[...]
    out_ref[...] = S * dis_ref[...] + b1_ref[...]


_final = pl.pallas_call(
    _final_body,
    out_shape=jax.ShapeDtypeStruct((N, H), jnp.float32),
)


@jax.jit
def kernel(edge_index, x, W0, b0, W1, b1):
    src = jnp.concatenate(
        [edge_index[0], jnp.zeros((PAD,), jnp.int32)])
    dst = jnp.concatenate(
        [edge_index[1],
         N + (jnp.arange(PAD, dtype=jnp.int32) % DUMMY_SPREAD)])
    pidx = (src | (dst << 16)).reshape(NW, CPT, CHUNK)

    degp = _deg_kernel(pidx)
    dis, g0 = _prologue(degp, x, W0)
    p0 = _segsum_kernel(pidx, g0)
    g1 = _mid(p0, g0, dis, b0.reshape(1, H), W1)
    p1 = _segsum_kernel(pidx, g1)
    out = _final(p1, g1, dis, b1.reshape(1, H))
    return out


# speedup vs baseline: 3.1566x; 3.1566x over previous
import functools

import jax
import jax.numpy as jnp
from jax import lax
from jax.experimental import pallas as pl
from jax.experimental.pallas import tpu as pltpu
from jax.experimental.pallas import tpu_sc as plsc

N = 10000
D = 128
H = 128
E = 320000

NC = 2
NS = 16
L = 16
NW = NC * NS

CHUNK = 128
CPT = 80
EPT = CPT * CHUNK
E_PAD = NW * EPT
PAD = E_PAD - E

ACC_ROWS = 10240
RPT = ACC_ROWS // NS
DUMMY_SPREAD = ACC_ROWS - N
DEG_W = 16

_mesh = plsc.VectorSubcoreMesh(core_axis_name="c", subcore_axis_name="s")


@functools.partial(
    pl.kernel,
    mesh=_mesh,
    out_type=jax.ShapeDtypeStruct((NC, ACC_ROWS, DEG_W), jnp.float32),
    scratch_types=[
        pltpu.VMEM((CPT, CHUNK), jnp.int32),
        pltpu.VMEM((CHUNK,), jnp.int32),
        pltpu.VMEM((CHUNK, DEG_W), jnp.float32),
        pltpu.VMEM_SHARED((ACC_ROWS, DEG_W), jnp.float32),
    ],
)
def _deg_kernel(pidx_hbm, out_hbm, pidx_v, didx_v, ones_v, acc_sh):
    c = lax.axis_index("c")
    s = lax.axis_index("s")
    w = c * NS + s

    pltpu.sync_copy(pidx_hbm.at[w], pidx_v)

    @pl.loop(0, CHUNK)
    def _(r):
        ones_v[r, pl.ds(0, L)] = jnp.zeros((L,), jnp.float32)

    @pl.loop(0, RPT, step=CHUNK)
    def _(r0):
        pltpu.sync_copy(ones_v, acc_sh.at[pl.ds(s * RPT + r0, CHUNK)])

    @pl.loop(0, CHUNK)
    def _(r):
        ones_v[r, pl.ds(0, L)] = jnp.ones((L,), jnp.float32)

    plsc.subcore_barrier()

    @pl.loop(0, CPT)
    def _(i):
        @pl.loop(0, CHUNK, step=L)
        def _(k):
            packed = pidx_v[i, pl.ds(k, L)]
            didx_v[pl.ds(k, L)] = lax.shift_right_logical(packed, 16)

        pltpu.sync_copy(ones_v, acc_sh.at[didx_v], add=True)

    plsc.subcore_barrier()
    pltpu.sync_copy(
        acc_sh.at[pl.ds(s * RPT, RPT)], out_hbm.at[c, pl.ds(s * RPT, RPT)]
    )


@functools.partial(
    pl.kernel,
    mesh=_mesh,
    out_type=jax.ShapeDtypeStruct((NC, ACC_ROWS, H), jnp.float32),
    scratch_types=[
        pltpu.VMEM((CPT, CHUNK), jnp.int32),
        pltpu.VMEM((CHUNK,), jnp.int32),
        pltpu.VMEM((CHUNK,), jnp.int32),
        pltpu.VMEM((CHUNK,), jnp.int32),
        pltpu.VMEM((CHUNK,), jnp.int32),
        pltpu.VMEM((CHUNK, H), jnp.float32),
        pltpu.VMEM((CHUNK, H), jnp.float32),
        pltpu.VMEM_SHARED((ACC_ROWS, H), jnp.float32),
        pltpu.SemaphoreType.DMA,
        pltpu.SemaphoreType.DMA,
    ],
)
def _segsum_kernel(pidx_hbm, g_hbm, out_hbm, pidx_v, sidx0_v, sidx1_v,
                   didx0_v, didx1_v, rows0_v, rows1_v, acc_sh, sem0, sem1):
    c = lax.axis_index("c")
    s = lax.axis_index("s")
    w = c * NS + s
    sidx = (sidx0_v, sidx1_v)
    didx = (didx0_v, didx1_v)
    rows = (rows0_v, rows1_v)
    sems = (sem0, sem1)

    pltpu.sync_copy(pidx_hbm.at[w], pidx_v)

    def unpack(cch, b):
        @pl.loop(0, CHUNK, step=L)
        def _(k):
            packed = pidx_v[cch, pl.ds(k, L)]
            sidx[b][pl.ds(k, L)] = lax.bitwise_and(packed, 0xFFFF)
            didx[b][pl.ds(k, L)] = lax.shift_right_logical(packed, 16)

    @pl.loop(0, CHUNK)
    def _(r):
        @pl.loop(0, H, step=L)
        def _(c0):
            rows0_v[r, pl.ds(c0, L)] = jnp.zeros((L,), jnp.float32)

    @pl.loop(0, RPT, step=CHUNK)
    def _(r0):
        pltpu.sync_copy(rows0_v, acc_sh.at[pl.ds(s * RPT + r0, CHUNK)])

    plsc.subcore_barrier()

    unpack(0, 0)
    pltpu.async_copy(g_hbm.at[sidx[0]], rows0_v, sem0)

    @pl.loop(0, CPT, step=2)
    def _(i):
        for b in range(2):
            cch = i + b
            pltpu.make_async_copy(
                g_hbm.at[sidx[b]], rows[b], sems[b]).wait()
            @pl.when(cch + 1 < CPT)
            def _():
                unpack(cch + 1, 1 - b)
                pltpu.async_copy(
                    g_hbm.at[sidx[1 - b]], rows[1 - b], sems[1 - b])
            pltpu.sync_copy(rows[b], acc_sh.at[didx[b]], add=True)

    plsc.subcore_barrier()
    pltpu.sync_copy(
        acc_sh.at[pl.ds(s * RPT, RPT)], out_hbm.at[c, pl.ds(s * RPT, RPT)]
    )


def _prologue_body(degp_ref, x_ref, w0_ref, dis_ref, g0_ref):
    degp = degp_ref[...]
    deg = jnp.sum(degp[0, :N, :] + degp[1, :N, :], axis=1) + 1.0
    dis = lax.rsqrt(deg)[:, None]
    h = jnp.dot(x_ref[...], w0_ref[...], preferred_element_type=jnp.float32)
    dis_ref[...] = dis
    g0_ref[...] = h * dis


_prologue = pl.pallas_call(
    _prologue_body,
    out_shape=[
        jax.ShapeDtypeStruct((N, 1), jnp.float32),
        jax.ShapeDtypeStruct((N, H), jnp.float32),
    ],
)


def _mid_body(p_ref, g0_ref, dis_ref, b0_ref, w1_ref, g1_ref):
    S = p_ref[0, :N, :] + p_ref[1, :N, :] + g0_ref[...]
    dis = dis_ref[...]
    h1 = jnp.maximum(S * dis + b0_ref[...], 0.0)
    g1_ref[...] = (
        jnp.dot(h1, w1_ref[...], preferred_element_type=jnp.float32) * dis
    )


_mid = pl.pallas_call(
    _mid_body,
    out_shape=jax.ShapeDtypeStruct((N, H), jnp.float32),
)


def _final_body(p_ref, g1_ref, dis_ref, b1_ref, out_ref):
    S = p_ref[0, :N, :] + p_ref[1, :N, :] + g1_ref[...]
    out_ref[...] = S * dis_ref[...] + b1_ref[...]


_final = pl.pallas_call(
    _final_body,
    out_shape=jax.ShapeDtypeStruct((N, H), jnp.float32),
)


@jax.jit
def kernel(edge_index, x, W0, b0, W1, b1):
    ppt = PAD // NW
    pad_src = (jnp.arange(NW * ppt, dtype=jnp.int32) * 131) % N
    pad_dst = N + (jnp.arange(NW * ppt, dtype=jnp.int32) % DUMMY_SPREAD)
    src = jnp.concatenate(
        [edge_index[0].reshape(NW, E // NW), pad_src.reshape(NW, ppt)], axis=1)
    dst = jnp.concatenate(
        [edge_index[1].reshape(NW, E // NW), pad_dst.reshape(NW, ppt)], axis=1)
    pidx = (src | (dst << 16)).reshape(NW, CPT, CHUNK)

    degp = _deg_kernel(pidx)
    dis, g0 = _prologue(degp, x, W0)
    p0 = _segsum_kernel(pidx, g0)
    g1 = _mid(p0, g0, dis, b0.reshape(1, H), W1)
    p1 = _segsum_kernel(pidx, g1)
    out = _final(p1, g1, dis, b1.reshape(1, H))
    return out
